# baseline (speedup 1.0000x reference)
import jax
import jax.numpy as jnp
from jax import lax
from jax.experimental import pallas as pl
from jax.experimental.pallas import tpu as pltpu
from jax.experimental.pallas import tpu_sc as plsc

N = 10000
E = 320000
D = 128
DH = D // 2

NC = 2
NS = 16
NW = NC * NS

CH = 128
CPT = 160
TOT_CH = NS * CPT
E_PAD = TOT_CH * CH
RPT = 632
N_PAD = RPT * NS
QPT = 160 // 2

K = 2
MC = 2

_TC_B = 1000


def _make_sc_pass(with_deg):
  def body(*refs):
    if with_deg:
      (y_hbm, src_hbm, dst_hbm, zacc_hbm, out_hbm, degp_hbm) = refs[:6]
      rest = refs[6:]
    else:
      (y_hbm, src_hbm, dst_hbm, zacc_hbm, out_hbm) = refs[:5]
      rest = refs[5:]
    src_pg, dst_pg = rest[0], rest[1]
    rows = rest[2:2 + K]
    dloc, acc = rest[2 + K], rest[3 + K]
    gsems = rest[4 + K:4 + 2 * K]
    ssems = rest[4 + 2 * K:4 + 3 * K]
    cid = lax.axis_index("c")
    sid = lax.axis_index("s")

    pltpu.sync_copy(zacc_hbm.at[pl.ds(sid * RPT, RPT)],
                    acc.at[pl.ds(sid * RPT, RPT)])
    if with_deg:
      def zloop(i, c):
        dloc[pl.ds(i * 16, 16)] = jnp.zeros((16,), jnp.float32)
        return c
      lax.fori_loop(0, N_PAD // 16, zloop, 0)

    plsc.subcore_barrier()

    pltpu.sync_copy(
        src_hbm.at[pl.ds((cid * TOT_CH + sid * CPT) // MC, QPT)], src_pg)
    pltpu.sync_copy(dst_hbm.at[pl.ds(sid * CPT, CPT)], dst_pg)

    ones16 = jnp.ones((16,), jnp.float32)

    def gather(h, b):
      pltpu.async_copy(y_hbm.at[src_pg.at[h]], rows[b], gsems[b])

    def gwait(b):
      pltpu.make_async_copy(y_hbm.at[src_pg.at[0]], rows[b], gsems[b]).wait()

    def sstart(h, b):
      for m in range(MC):
        pltpu.async_copy(rows[b].at[pl.ds(m * CH, CH)],
                         acc.at[dst_pg.at[MC * h + m]], ssems[b], add=True)

    def swait(b):
      for m in range(MC):
        pltpu.make_async_copy(rows[b].at[pl.ds(m * CH, CH)],
                              acc.at[dst_pg.at[0]], ssems[b]).wait()

    def deg(h):
      if not with_deg:
        return
      @pl.when(jnp.logical_xor(cid == 1, h < QPT // 2))
      def _():
        for m in range(MC):
          for k in range(CH // 16):
            idx = dst_pg[MC * h + m, pl.ds(k * 16, 16)]
            plsc.addupdate_scatter(dloc, [idx], ones16)

    for b in range(K):
      gather(b, b)

    def group(g, carry):
      h0 = K * g
      for b in range(K):
        gwait(b)
        sstart(h0 + b, b)
      for b in range(K):
        deg(h0 + b)
        swait(b)

        @pl.when(g < QPT // K - 1)
        def _():
          gather(h0 + K + b, b)
      return carry

    lax.fori_loop(0, QPT // K, group, 0)

    if with_deg:
      pltpu.sync_copy(dloc, degp_hbm.at[pl.ds((cid * NS + sid) * N_PAD,
                                              N_PAD)])

    plsc.subcore_barrier()

    pltpu.sync_copy(acc.at[pl.ds(sid * RPT, RPT)],
                    out_hbm.at[pl.ds(cid * N_PAD + sid * RPT, RPT)])

  def run(y, src2, dst, zacc):
    mesh = plsc.VectorSubcoreMesh(core_axis_name="c", subcore_axis_name="s")
    out_type = (jax.ShapeDtypeStruct((NC * N_PAD, DH), jnp.float32),)
    if with_deg:
      out_type += (jax.ShapeDtypeStruct((NW * N_PAD,), jnp.float32),)
    scratch = (
        [pltpu.VMEM((QPT, MC * CH), jnp.int32),
         pltpu.VMEM((CPT, CH), jnp.int32)]
        + [pltpu.VMEM((MC * CH, DH), jnp.float32)] * K
        + [pltpu.VMEM((N_PAD,), jnp.float32),
           pltpu.VMEM_SHARED((N_PAD, DH), jnp.float32)]
        + [pltpu.SemaphoreType.DMA] * (2 * K)
    )
    fn = pl.kernel(
        body, out_type, mesh=mesh, scratch_types=scratch,
        compiler_params=pltpu.CompilerParams(
            needs_layout_passes=False, use_tc_tiling_on_sc=False))
    return fn(y, src2, dst, zacc)

  return run


_sc_pass_deg = _make_sc_pass(True)
_sc_pass_nodeg = _make_sc_pass(False)


def _dotT(a, w):
  return lax.dot_general(a, w, (((1,), (1,)), ((), ())),
                         preferred_element_type=jnp.float32)


def _tc_pre_body(x_ref, wl_ref, wr_ref, b_ref, y_ref, z_ref):
  x = x_ref[...]
  y = _dotT(x, wl_ref[...])
  y_ref[0] = y[:, :DH]
  y_ref[1] = y[:, DH:]
  z_ref[...] = _dotT(x, wr_ref[...]) + b_ref[...]


def _tc_mid_body(p_ref, dt_ref, z1_ref, wl_ref, wr_ref, b_ref, y_ref, z_ref):
  s = jnp.concatenate([p_ref[0], p_ref[1]], axis=1)
  d = jnp.maximum(jnp.sum(dt_ref[...], axis=1, keepdims=True), 1.0)
  h = jnp.maximum(s / d + z1_ref[...], 0.0)
  y = _dotT(h, wl_ref[...])
  y_ref[0] = y[:, :DH]
  y_ref[1] = y[:, DH:]
  z_ref[...] = _dotT(h, wr_ref[...]) + b_ref[...]


def _tc_post_body(p_ref, dt_ref, z2_ref, o_ref):
  s = jnp.concatenate([p_ref[0], p_ref[1]], axis=1)
  d = jnp.maximum(jnp.sum(dt_ref[...], axis=1, keepdims=True), 1.0)
  u = s / d + z2_ref[...]
  nrm = jnp.sqrt(jnp.sum(u * u, axis=1, keepdims=True))
  o_ref[...] = u / jnp.maximum(nrm, 1e-12)


def _row_spec(b):
  return pl.BlockSpec((b, D), lambda i: (i, 0))


def _half_spec(b):
  return pl.BlockSpec((NC, b, DH), lambda i: (0, i, 0))


def _full_spec(shape):
  n = len(shape)
  return pl.BlockSpec(shape, lambda i: (0,) * n)


def _tc_pre(x, wl, wr, b):
  return pl.pallas_call(
      _tc_pre_body,
      grid=(N // _TC_B,),
      in_specs=[_row_spec(_TC_B), _full_spec((D, D)), _full_spec((D, D)),
                _full_spec((1, D))],
      out_specs=[_half_spec(_TC_B), _row_spec(_TC_B)],
      out_shape=[jax.ShapeDtypeStruct((NC, N_PAD, DH), jnp.float32),
                 jax.ShapeDtypeStruct((N, D), jnp.float32)],
  )(x, wl, wr, b)


def _tc_mid(p, dt, z1, wl, wr, b):
  return pl.pallas_call(
      _tc_mid_body,
      grid=(N // _TC_B,),
      in_specs=[_half_spec(_TC_B),
                pl.BlockSpec((_TC_B, NW), lambda i: (i, 0)),
                _row_spec(_TC_B), _full_spec((D, D)), _full_spec((D, D)),
                _full_spec((1, D))],
      out_specs=[_half_spec(_TC_B), _row_spec(_TC_B)],
      out_shape=[jax.ShapeDtypeStruct((NC, N_PAD, DH), jnp.float32),
                 jax.ShapeDtypeStruct((N, D), jnp.float32)],
  )(p, dt, z1, wl, wr, b)


def _tc_post(p, dt, z2):
  return pl.pallas_call(
      _tc_post_body,
      grid=(N // _TC_B,),
      in_specs=[_half_spec(_TC_B),
                pl.BlockSpec((_TC_B, NW), lambda i: (i, 0)),
                _row_spec(_TC_B)],
      out_specs=_row_spec(_TC_B),
      out_shape=jax.ShapeDtypeStruct((N, D), jnp.float32),
  )(p, dt, z2)


@jax.jit
def kernel(x, edge_index, W_l1, b_l1, W_r1, W_l2, b_l2, W_r2):
  srcp = jnp.concatenate([edge_index[0], jnp.zeros((E_PAD - E,), jnp.int32)])
  src2 = jnp.concatenate([srcp, srcp + N_PAD]).reshape(NC * TOT_CH // MC, MC * CH)
  dst = jnp.concatenate(
      [edge_index[1], jnp.full((E_PAD - E,), N, jnp.int32)]
  ).reshape(TOT_CH, CH)
  zacc = jnp.zeros((N_PAD, DH), jnp.float32)

  yh1, z1 = _tc_pre(x, W_l1, W_r1, b_l1.reshape(1, D))
  p1, degp = _sc_pass_deg(yh1.reshape(NC * N_PAD, DH), src2, dst, zacc)
  dt = degp.reshape(NW, N_PAD).T[:N]
  yh2, z2 = _tc_mid(p1.reshape(NC, N_PAD, DH), dt, z1, W_l2, W_r2,
                    b_l2.reshape(1, D))
  (p2,) = _sc_pass_nodeg(yh2.reshape(NC * N_PAD, DH), src2, dst, zacc)
  return _tc_post(p2.reshape(NC, N_PAD, DH), dt, z2)


# speedup vs baseline: 1.0541x; 1.0541x over previous
import jax
import jax.numpy as jnp
from jax import lax
from jax.experimental import pallas as pl
from jax.experimental.pallas import tpu as pltpu
from jax.experimental.pallas import tpu_sc as plsc

N = 10000
E = 320000
D = 128
DH = D // 2

NC = 2
NS = 16
NW = NC * NS

CH = 128
CPT = 160
TOT_CH = NS * CPT
E_PAD = TOT_CH * CH
RPT = 632
N_PAD = RPT * NS
QPT = 160 // 2

KD = 4
KN = 5

_TC_B = 1000


def _make_sc_pass(with_deg):
  def body(*refs):
    K = KD if with_deg else KN
    if with_deg:
      (y_hbm, src_hbm, dst_hbm, zacc_hbm, out_hbm, degp_hbm) = refs[:6]
      rest = refs[6:]
    else:
      (y_hbm, src_hbm, dst_hbm, zacc_hbm, out_hbm) = refs[:5]
      rest = refs[5:]
    src_pg, dst_pg = rest[0], rest[1]
    rows = rest[2:2 + K]
    if with_deg:
      dloc = rest[2 + K]
      rest = rest[3 + K:]
    else:
      dloc = None
      rest = rest[2 + K:]
    acc = rest[0]
    gsems = rest[1:1 + K]
    ssems = rest[1 + K:1 + 2 * K]
    cid = lax.axis_index("c")
    sid = lax.axis_index("s")

    pltpu.sync_copy(zacc_hbm.at[pl.ds(sid * RPT, RPT)],
                    acc.at[pl.ds(sid * RPT, RPT)])
    if with_deg:
      def zloop(i, c):
        dloc[pl.ds(i * 16, 16)] = jnp.zeros((16,), jnp.float32)
        return c
      lax.fori_loop(0, N_PAD // 16, zloop, 0)

    plsc.subcore_barrier()

    pltpu.sync_copy(src_hbm.at[pl.ds(cid * TOT_CH + sid * CPT, CPT)], src_pg)
    pltpu.sync_copy(dst_hbm.at[pl.ds(sid * CPT, CPT)], dst_pg)

    ones16 = jnp.ones((16,), jnp.float32)

    def gather(j, b):
      pltpu.async_copy(y_hbm.at[src_pg.at[j]], rows[b], gsems[b])

    def gwait(b):
      pltpu.make_async_copy(y_hbm.at[src_pg.at[0]], rows[b], gsems[b]).wait()

    def sstart(j, b):
      pltpu.async_copy(rows[b], acc.at[dst_pg.at[j]], ssems[b], add=True)

    def swait(b):
      pltpu.make_async_copy(rows[b], acc.at[dst_pg.at[0]], ssems[b]).wait()

    def deg(j):
      if not with_deg:
        return
      @pl.when(jnp.logical_xor(cid == 1, j < CPT // 2))
      def _():
        for k in range(CH // 16):
          idx = dst_pg[j, pl.ds(k * 16, 16)]
          plsc.addupdate_scatter(dloc, [idx], ones16)

    for b in range(K):
      gather(b, b)

    def group(g, carry):
      j0 = K * g
      for b in range(K):
        gwait(b)
        sstart(j0 + b, b)
      for b in range(K):
        deg(j0 + b)
        swait(b)

        @pl.when(g < CPT // K - 1)
        def _():
          gather(j0 + K + b, b)
      return carry

    lax.fori_loop(0, CPT // K, group, 0)

    if with_deg:
      pltpu.sync_copy(dloc, degp_hbm.at[pl.ds((cid * NS + sid) * N_PAD,
                                              N_PAD)])

    plsc.subcore_barrier()

    pltpu.sync_copy(acc.at[pl.ds(sid * RPT, RPT)],
                    out_hbm.at[pl.ds(cid * N_PAD + sid * RPT, RPT)])

  def run(y, src2, dst, zacc):
    mesh = plsc.VectorSubcoreMesh(core_axis_name="c", subcore_axis_name="s")
    out_type = (jax.ShapeDtypeStruct((NC * N_PAD, DH), jnp.float32),)
    if with_deg:
      out_type += (jax.ShapeDtypeStruct((NW * N_PAD,), jnp.float32),)
    K = KD if with_deg else KN
    scratch = (
        [pltpu.VMEM((CPT, CH), jnp.int32),
         pltpu.VMEM((CPT, CH), jnp.int32)]
        + [pltpu.VMEM((CH, DH), jnp.float32)] * K
        + ([pltpu.VMEM((N_PAD,), jnp.float32)]
           if with_deg else [])
        + [pltpu.VMEM_SHARED((N_PAD, DH), jnp.float32)]
        + [pltpu.SemaphoreType.DMA] * (2 * K)
    )
    fn = pl.kernel(
        body, out_type, mesh=mesh, scratch_types=scratch,
        compiler_params=pltpu.CompilerParams(
            needs_layout_passes=False, use_tc_tiling_on_sc=False))
    return fn(y, src2, dst, zacc)

  return run


_sc_pass_deg = _make_sc_pass(True)
_sc_pass_nodeg = _make_sc_pass(False)


def _dotT(a, w):
  return lax.dot_general(a, w, (((1,), (1,)), ((), ())),
                         preferred_element_type=jnp.float32)


def _tc_pre_body(x_ref, wl_ref, wr_ref, b_ref, y_ref, z_ref):
  x = x_ref[...]
  y = _dotT(x, wl_ref[...])
  y_ref[0] = y[:, :DH]
  y_ref[1] = y[:, DH:]
  z_ref[...] = _dotT(x, wr_ref[...]) + b_ref[...]


def _tc_mid_body(p_ref, dt_ref, z1_ref, wl_ref, wr_ref, b_ref, y_ref, z_ref):
  s = jnp.concatenate([p_ref[0], p_ref[1]], axis=1)
  d = jnp.maximum(jnp.sum(dt_ref[...], axis=1, keepdims=True), 1.0)
  h = jnp.maximum(s / d + z1_ref[...], 0.0)
  y = _dotT(h, wl_ref[...])
  y_ref[0] = y[:, :DH]
  y_ref[1] = y[:, DH:]
  z_ref[...] = _dotT(h, wr_ref[...]) + b_ref[...]


def _tc_post_body(p_ref, dt_ref, z2_ref, o_ref):
  s = jnp.concatenate([p_ref[0], p_ref[1]], axis=1)
  d = jnp.maximum(jnp.sum(dt_ref[...], axis=1, keepdims=True), 1.0)
  u = s / d + z2_ref[...]
  nrm = jnp.sqrt(jnp.sum(u * u, axis=1, keepdims=True))
  o_ref[...] = u / jnp.maximum(nrm, 1e-12)


def _row_spec(b):
  return pl.BlockSpec((b, D), lambda i: (i, 0))


def _half_spec(b):
  return pl.BlockSpec((NC, b, DH), lambda i: (0, i, 0))


def _full_spec(shape):
  n = len(shape)
  return pl.BlockSpec(shape, lambda i: (0,) * n)


def _tc_pre(x, wl, wr, b):
  return pl.pallas_call(
      _tc_pre_body,
      grid=(N // _TC_B,),
      in_specs=[_row_spec(_TC_B), _full_spec((D, D)), _full_spec((D, D)),
                _full_spec((1, D))],
      out_specs=[_half_spec(_TC_B), _row_spec(_TC_B)],
      out_shape=[jax.ShapeDtypeStruct((NC, N_PAD, DH), jnp.float32),
                 jax.ShapeDtypeStruct((N, D), jnp.float32)],
  )(x, wl, wr, b)


def _tc_mid(p, dt, z1, wl, wr, b):
  return pl.pallas_call(
      _tc_mid_body,
      grid=(N // _TC_B,),
      in_specs=[_half_spec(_TC_B),
                pl.BlockSpec((_TC_B, NW), lambda i: (i, 0)),
                _row_spec(_TC_B), _full_spec((D, D)), _full_spec((D, D)),
                _full_spec((1, D))],
      out_specs=[_half_spec(_TC_B), _row_spec(_TC_B)],
      out_shape=[jax.ShapeDtypeStruct((NC, N_PAD, DH), jnp.float32),
                 jax.ShapeDtypeStruct((N, D), jnp.float32)],
  )(p, dt, z1, wl, wr, b)


def _tc_post(p, dt, z2):
  return pl.pallas_call(
      _tc_post_body,
      grid=(N // _TC_B,),
      in_specs=[_half_spec(_TC_B),
                pl.BlockSpec((_TC_B, NW), lambda i: (i, 0)),
                _row_spec(_TC_B)],
      out_specs=_row_spec(_TC_B),
      out_shape=jax.ShapeDtypeStruct((N, D), jnp.float32),
  )(p, dt, z2)


@jax.jit
def kernel(x, edge_index, W_l1, b_l1, W_r1, W_l2, b_l2, W_r2):
  srcp = jnp.concatenate([edge_index[0], jnp.zeros((E_PAD - E,), jnp.int32)])
  src2 = jnp.concatenate([srcp, srcp + N_PAD]).reshape(NC * TOT_CH, CH)
  dst = jnp.concatenate(
      [edge_index[1], jnp.full((E_PAD - E,), N, jnp.int32)]
  ).reshape(TOT_CH, CH)
  zacc = jnp.zeros((N_PAD, DH), jnp.float32)

  yh1, z1 = _tc_pre(x, W_l1, W_r1, b_l1.reshape(1, D))
  p1, degp = _sc_pass_deg(yh1.reshape(NC * N_PAD, DH), src2, dst, zacc)
  dt = degp.reshape(NW, N_PAD).T[:N]
  yh2, z2 = _tc_mid(p1.reshape(NC, N_PAD, DH), dt, z1, W_l2, W_r2,
                    b_l2.reshape(1, D))
  (p2,) = _sc_pass_nodeg(yh2.reshape(NC * N_PAD, DH), src2, dst, zacc)
  return _tc_post(p2.reshape(NC, N_PAD, DH), dt, z2)


# speedup vs baseline: 1.0550x; 1.0009x over previous
import jax
import jax.numpy as jnp
from jax import lax
from jax.experimental import pallas as pl
from jax.experimental.pallas import tpu as pltpu
from jax.experimental.pallas import tpu_sc as plsc

N = 10000
E = 320000
D = 128
DH = D // 2

NC = 2
NS = 16
NW = NC * NS

CH = 128
CPT = 160
TOT_CH = NS * CPT
E_PAD = TOT_CH * CH
RPT = 632
N_PAD = RPT * NS
QPT = 160 // 2

KD = 4
KN = 5

_TC_B = 1000


def _make_sc_pass(with_deg):
  def body(*refs):
    K = KD if with_deg else KN
    if with_deg:
      (y_hbm, src_hbm, dst_hbm, zacc_hbm, out_hbm, degp_hbm) = refs[:6]
      rest = refs[6:]
    else:
      (y_hbm, src_hbm, dst_hbm, zacc_hbm, out_hbm) = refs[:5]
      rest = refs[5:]
    src_pg, dst_pg = rest[0], rest[1]
    rows = rest[2:2 + K]
    if with_deg:
      dloc = rest[2 + K]
      rest = rest[3 + K:]
    else:
      dloc = None
      rest = rest[2 + K:]
    acc = rest[0]
    gsems = rest[1:1 + K]
    ssems = rest[1 + K:1 + 2 * K]
    cid = lax.axis_index("c")
    sid = lax.axis_index("s")

    pltpu.sync_copy(zacc_hbm.at[pl.ds(sid * RPT, RPT)],
                    acc.at[pl.ds(sid * RPT, RPT)])
    if with_deg:
      def zloop(i, c):
        dloc[pl.ds(i * 16, 16)] = jnp.zeros((16,), jnp.float32)
        return c
      lax.fori_loop(0, N_PAD // 16, zloop, 0)

    plsc.subcore_barrier()

    pltpu.sync_copy(src_hbm.at[pl.ds(cid * TOT_CH + sid * CPT, CPT)], src_pg)
    pltpu.sync_copy(dst_hbm.at[pl.ds(sid * CPT, CPT)], dst_pg)

    ones16 = jnp.ones((16,), jnp.float32)

    def gather(j, b):
      pltpu.async_copy(y_hbm.at[src_pg.at[j]], rows[b], gsems[b])

    def gwait(b):
      pltpu.make_async_copy(y_hbm.at[src_pg.at[0]], rows[b], gsems[b]).wait()

    def sstart(j, b):
      pltpu.async_copy(rows[b], acc.at[dst_pg.at[j]], ssems[b], add=True)

    def swait(b):
      pltpu.make_async_copy(rows[b], acc.at[dst_pg.at[0]], ssems[b]).wait()

    def deg(j):
      if not with_deg:
        return
      @pl.when(jnp.logical_xor(cid == 1, j < CPT // 2))
      def _():
        for k in range(CH // 16):
          idx = dst_pg[j, pl.ds(k * 16, 16)]
          plsc.addupdate_scatter(dloc, [idx], ones16)

    for b in range(K):
      gather(b, b)

    def group(g, carry):
      j0 = K * g
      for b in range(K):
        gwait(b)
        sstart(j0 + b, b)
      for b in range(K):
        swait(b)
        gather(j0 + K + b, b)
      for b in range(K):
        deg(j0 + b)
      return carry

    lax.fori_loop(0, CPT // K - 1, group, 0)

    j0 = CPT - K
    for b in range(K):
      gwait(b)
      sstart(j0 + b, b)
    for b in range(K):
      deg(j0 + b)
      swait(b)

    if with_deg:
      pltpu.sync_copy(dloc, degp_hbm.at[pl.ds((cid * NS + sid) * N_PAD,
                                              N_PAD)])

    plsc.subcore_barrier()

    pltpu.sync_copy(acc.at[pl.ds(sid * RPT, RPT)],
                    out_hbm.at[pl.ds(cid * N_PAD + sid * RPT, RPT)])

  def run(y, src2, dst, zacc):
    mesh = plsc.VectorSubcoreMesh(core_axis_name="c", subcore_axis_name="s")
    out_type = (jax.ShapeDtypeStruct((NC * N_PAD, DH), jnp.float32),)
    if with_deg:
      out_type += (jax.ShapeDtypeStruct((NW * N_PAD,), jnp.float32),)
    K = KD if with_deg else KN
    scratch = (
        [pltpu.VMEM((CPT, CH), jnp.int32),
         pltpu.VMEM((CPT, CH), jnp.int32)]
        + [pltpu.VMEM((CH, DH), jnp.float32)] * K
        + ([pltpu.VMEM((N_PAD,), jnp.float32)]
           if with_deg else [])
        + [pltpu.VMEM_SHARED((N_PAD, DH), jnp.float32)]
        + [pltpu.SemaphoreType.DMA] * (2 * K)
    )
    fn = pl.kernel(
        body, out_type, mesh=mesh, scratch_types=scratch,
        compiler_params=pltpu.CompilerParams(
            needs_layout_passes=False, use_tc_tiling_on_sc=False))
    return fn(y, src2, dst, zacc)

  return run


_sc_pass_deg = _make_sc_pass(True)
_sc_pass_nodeg = _make_sc_pass(False)


def _dotT(a, w):
  return lax.dot_general(a, w, (((1,), (1,)), ((), ())),
                         preferred_element_type=jnp.float32)


def _tc_pre_body(x_ref, wl_ref, wr_ref, b_ref, y_ref, z_ref):
  x = x_ref[...]
  y = _dotT(x, wl_ref[...])
  y_ref[0] = y[:, :DH]
  y_ref[1] = y[:, DH:]
  z_ref[...] = _dotT(x, wr_ref[...]) + b_ref[...]


def _tc_mid_body(p_ref, dt_ref, z1_ref, wl_ref, wr_ref, b_ref, y_ref, z_ref):
  s = jnp.concatenate([p_ref[0], p_ref[1]], axis=1)
  d = jnp.maximum(jnp.sum(dt_ref[...], axis=1, keepdims=True), 1.0)
  h = jnp.maximum(s / d + z1_ref[...], 0.0)
  y = _dotT(h, wl_ref[...])
  y_ref[0] = y[:, :DH]
  y_ref[1] = y[:, DH:]
  z_ref[...] = _dotT(h, wr_ref[...]) + b_ref[...]


def _tc_post_body(p_ref, dt_ref, z2_ref, o_ref):
  s = jnp.concatenate([p_ref[0], p_ref[1]], axis=1)
  d = jnp.maximum(jnp.sum(dt_ref[...], axis=1, keepdims=True), 1.0)
  u = s / d + z2_ref[...]
  nrm = jnp.sqrt(jnp.sum(u * u, axis=1, keepdims=True))
  o_ref[...] = u / jnp.maximum(nrm, 1e-12)


def _row_spec(b):
  return pl.BlockSpec((b, D), lambda i: (i, 0))


def _half_spec(b):
  return pl.BlockSpec((NC, b, DH), lambda i: (0, i, 0))


def _full_spec(shape):
  n = len(shape)
  return pl.BlockSpec(shape, lambda i: (0,) * n)


def _tc_pre(x, wl, wr, b):
  return pl.pallas_call(
      _tc_pre_body,
      grid=(N // _TC_B,),
      in_specs=[_row_spec(_TC_B), _full_spec((D, D)), _full_spec((D, D)),
                _full_spec((1, D))],
      out_specs=[_half_spec(_TC_B), _row_spec(_TC_B)],
      out_shape=[jax.ShapeDtypeStruct((NC, N_PAD, DH), jnp.float32),
                 jax.ShapeDtypeStruct((N, D), jnp.float32)],
  )(x, wl, wr, b)


def _tc_mid(p, dt, z1, wl, wr, b):
  return pl.pallas_call(
      _tc_mid_body,
      grid=(N // _TC_B,),
      in_specs=[_half_spec(_TC_B),
                pl.BlockSpec((_TC_B, NW), lambda i: (i, 0)),
                _row_spec(_TC_B), _full_spec((D, D)), _full_spec((D, D)),
                _full_spec((1, D))],
      out_specs=[_half_spec(_TC_B), _row_spec(_TC_B)],
      out_shape=[jax.ShapeDtypeStruct((NC, N_PAD, DH), jnp.float32),
                 jax.ShapeDtypeStruct((N, D), jnp.float32)],
  )(p, dt, z1, wl, wr, b)


def _tc_post(p, dt, z2):
  return pl.pallas_call(
      _tc_post_body,
      grid=(N // _TC_B,),
      in_specs=[_half_spec(_TC_B),
                pl.BlockSpec((_TC_B, NW), lambda i: (i, 0)),
                _row_spec(_TC_B)],
      out_specs=_row_spec(_TC_B),
      out_shape=jax.ShapeDtypeStruct((N, D), jnp.float32),
  )(p, dt, z2)


@jax.jit
def kernel(x, edge_index, W_l1, b_l1, W_r1, W_l2, b_l2, W_r2):
  srcp = jnp.concatenate([edge_index[0], jnp.zeros((E_PAD - E,), jnp.int32)])
  src2 = jnp.concatenate([srcp, srcp + N_PAD]).reshape(NC * TOT_CH, CH)
  dst = jnp.concatenate(
      [edge_index[1], jnp.full((E_PAD - E,), N, jnp.int32)]
  ).reshape(TOT_CH, CH)
  zacc = jnp.zeros((N_PAD, DH), jnp.float32)

  yh1, z1 = _tc_pre(x, W_l1, W_r1, b_l1.reshape(1, D))
  p1, degp = _sc_pass_deg(yh1.reshape(NC * N_PAD, DH), src2, dst, zacc)
  dt = degp.reshape(NW, N_PAD).T[:N]
  yh2, z2 = _tc_mid(p1.reshape(NC, N_PAD, DH), dt, z1, W_l2, W_r2,
                    b_l2.reshape(1, D))
  (p2,) = _sc_pass_nodeg(yh2.reshape(NC * N_PAD, DH), src2, dst, zacc)
  return _tc_post(p2.reshape(NC, N_PAD, DH), dt, z2)


# speedup vs baseline: 1.0554x; 1.0004x over previous
import jax
import jax.numpy as jnp
from jax import lax
from jax.experimental import pallas as pl
from jax.experimental.pallas import tpu as pltpu
from jax.experimental.pallas import tpu_sc as plsc

N = 10000
E = 320000
D = 128
DH = D // 2

NC = 2
NS = 16
NW = NC * NS

CH = 128
CPT = 160
TOT_CH = NS * CPT
E_PAD = TOT_CH * CH
RPT = 632
N_PAD = RPT * NS

KD = 4
KN = 5

_TC_B = 1000


def _make_sc_pass(with_deg):
  def body(*refs):
    K = KD if with_deg else KN
    if with_deg:
      (y_hbm, src_hbm, dst_hbm, zacc_hbm, out_hbm, degp_hbm) = refs[:6]
      rest = refs[6:]
    else:
      (y_hbm, src_hbm, dst_hbm, zacc_hbm, out_hbm) = refs[:5]
      rest = refs[5:]
    src_pg, dst_pg = rest[0], rest[1]
    rows = rest[2:2 + K]
    if with_deg:
      dloc = rest[2 + K]
      rest = rest[3 + K:]
    else:
      dloc = None
      rest = rest[2 + K:]
    acc = rest[0]
    gsems = rest[1:1 + K]
    ssems = rest[1 + K:1 + 2 * K]
    cid = lax.axis_index("c")
    sid = lax.axis_index("s")

    pltpu.sync_copy(zacc_hbm.at[pl.ds(sid * RPT, RPT)],
                    acc.at[pl.ds(sid * RPT, RPT)])
    if with_deg:
      def zloop(i, c):
        dloc[pl.ds(i * 16, 16)] = jnp.zeros((16,), jnp.float32)
        return c
      lax.fori_loop(0, N_PAD // 16, zloop, 0)

    plsc.subcore_barrier()

    pltpu.sync_copy(src_hbm.at[pl.ds(cid * TOT_CH + sid * CPT, CPT)], src_pg)
    pltpu.sync_copy(dst_hbm.at[pl.ds(sid * CPT, CPT)], dst_pg)

    ones16 = jnp.ones((16,), jnp.float32)

    def gather(j, b):
      pltpu.async_copy(y_hbm.at[src_pg.at[j]], rows[b], gsems[b])

    def gwait(b):
      pltpu.make_async_copy(y_hbm.at[src_pg.at[0]], rows[b], gsems[b]).wait()

    def sstart(j, b):
      pltpu.async_copy(rows[b], acc.at[dst_pg.at[j]], ssems[b], add=True)

    def swait(b):
      pltpu.make_async_copy(rows[b], acc.at[dst_pg.at[0]], ssems[b]).wait()

    def deg(j):
      if not with_deg:
        return
      @pl.when(jnp.logical_xor(cid == 1, j < CPT // 2))
      def _():
        for k in range(CH // 16):
          idx = dst_pg[j, pl.ds(k * 16, 16)]
          plsc.addupdate_scatter(dloc, [idx], ones16)

    for b in range(K):
      gather(b, b)

    def group(g, carry):
      j0 = K * g
      for b in range(K):
        gwait(b)
        sstart(j0 + b, b)
      for b in range(K):
        swait(b)
        gather(j0 + K + b, b)
      for b in range(K):
        deg(j0 + b)
      return carry

    lax.fori_loop(0, CPT // K - 1, group, 0)

    j0 = CPT - K
    for b in range(K):
      gwait(b)
      sstart(j0 + b, b)
    for b in range(K):
      deg(j0 + b)
      swait(b)

    if with_deg:
      pltpu.sync_copy(dloc, degp_hbm.at[pl.ds((cid * NS + sid) * N_PAD,
                                              N_PAD)])

    plsc.subcore_barrier()

    pltpu.sync_copy(acc.at[pl.ds(sid * RPT, RPT)],
                    out_hbm.at[pl.ds(cid * N_PAD + sid * RPT, RPT)])

  def run(y, src2, dst, zacc):
    mesh = plsc.VectorSubcoreMesh(core_axis_name="c", subcore_axis_name="s")
    out_type = (jax.ShapeDtypeStruct((NC * N_PAD, DH), jnp.float32),)
    if with_deg:
      out_type += (jax.ShapeDtypeStruct((NW * N_PAD,), jnp.float32),)
    K = KD if with_deg else KN
    scratch = (
        [pltpu.VMEM((CPT, CH), jnp.int32),
         pltpu.VMEM((CPT, CH), jnp.int32)]
        + [pltpu.VMEM((CH, DH), jnp.float32)] * K
        + ([pltpu.VMEM((N_PAD,), jnp.float32)]
           if with_deg else [])
        + [pltpu.VMEM_SHARED((N_PAD, DH), jnp.float32)]
        + [pltpu.SemaphoreType.DMA] * (2 * K)
    )
    fn = pl.kernel(
        body, out_type, mesh=mesh, scratch_types=scratch,
        compiler_params=pltpu.CompilerParams(
            needs_layout_passes=False, use_tc_tiling_on_sc=False))
    return fn(y, src2, dst, zacc)

  return run


_sc_pass_deg = _make_sc_pass(True)
_sc_pass_nodeg = _make_sc_pass(False)


def _dotT(a, w):
  return lax.dot_general(a, w, (((1,), (1,)), ((), ())),
                         preferred_element_type=jnp.float32)


def _tc_pre_body(x_ref, wl_ref, wr_ref, b_ref, y_ref, z_ref):
  x = x_ref[...]
  y = _dotT(x, wl_ref[...])
  y_ref[0] = y[:, :DH]
  y_ref[1] = y[:, DH:]
  z_ref[...] = _dotT(x, wr_ref[...]) + b_ref[...]


def _tc_mid_body(p_ref, dt_ref, z1_ref, wl_ref, wr_ref, b_ref, y_ref, z_ref):
  s = jnp.concatenate([p_ref[0], p_ref[1]], axis=1)
  d = jnp.maximum(jnp.sum(dt_ref[...], axis=1, keepdims=True), 1.0)
  h = jnp.maximum(s / d + z1_ref[...], 0.0)
  y = _dotT(h, wl_ref[...])
  y_ref[0] = y[:, :DH]
  y_ref[1] = y[:, DH:]
  z_ref[...] = _dotT(h, wr_ref[...]) + b_ref[...]


def _tc_post_body(p_ref, dt_ref, z2_ref, o_ref):
  s = jnp.concatenate([p_ref[0], p_ref[1]], axis=1)
  d = jnp.maximum(jnp.sum(dt_ref[...], axis=1, keepdims=True), 1.0)
  u = s / d + z2_ref[...]
  nrm = jnp.sqrt(jnp.sum(u * u, axis=1, keepdims=True))
  o_ref[...] = u / jnp.maximum(nrm, 1e-12)


def _row_spec(b):
  return pl.BlockSpec((b, D), lambda i: (i, 0))


def _half_spec(b):
  return pl.BlockSpec((NC, b, DH), lambda i: (0, i, 0))


def _full_spec(shape):
  n = len(shape)
  return pl.BlockSpec(shape, lambda i: (0,) * n)


def _tc_pre(x, wl, wr, b):
  return pl.pallas_call(
      _tc_pre_body,
      grid=(N // _TC_B,),
      in_specs=[_row_spec(_TC_B), _full_spec((D, D)), _full_spec((D, D)),
                _full_spec((1, D))],
      out_specs=[_half_spec(_TC_B), _row_spec(_TC_B)],
      out_shape=[jax.ShapeDtypeStruct((NC, N_PAD, DH), jnp.float32),
                 jax.ShapeDtypeStruct((N, D), jnp.float32)],
  )(x, wl, wr, b)


def _tc_mid(p, dt, z1, wl, wr, b):
  return pl.pallas_call(
      _tc_mid_body,
      grid=(N // _TC_B,),
      in_specs=[_half_spec(_TC_B),
                pl.BlockSpec((_TC_B, NW), lambda i: (i, 0)),
                _row_spec(_TC_B), _full_spec((D, D)), _full_spec((D, D)),
                _full_spec((1, D))],
      out_specs=[_half_spec(_TC_B), _row_spec(_TC_B)],
      out_shape=[jax.ShapeDtypeStruct((NC, N_PAD, DH), jnp.float32),
                 jax.ShapeDtypeStruct((N, D), jnp.float32)],
  )(p, dt, z1, wl, wr, b)


def _tc_post(p, dt, z2):
  return pl.pallas_call(
      _tc_post_body,
      grid=(N // _TC_B,),
      in_specs=[_half_spec(_TC_B),
                pl.BlockSpec((_TC_B, NW), lambda i: (i, 0)),
                _row_spec(_TC_B)],
      out_specs=_row_spec(_TC_B),
      out_shape=jax.ShapeDtypeStruct((N, D), jnp.float32),
  )(p, dt, z2)


@jax.jit
def kernel(x, edge_index, W_l1, b_l1, W_r1, W_l2, b_l2, W_r2):
  srcp = jnp.concatenate([edge_index[0], jnp.zeros((E_PAD - E,), jnp.int32)])
  src2 = jnp.concatenate([srcp, srcp + N_PAD]).reshape(NC * TOT_CH, CH)
  dst = jnp.concatenate(
      [edge_index[1], jnp.full((E_PAD - E,), N, jnp.int32)]
  ).reshape(TOT_CH, CH)
  zacc = jnp.zeros((N_PAD, DH), jnp.float32)

  yh1, z1 = _tc_pre(x, W_l1, W_r1, b_l1.reshape(1, D))
  p1, degp = _sc_pass_deg(yh1.reshape(NC * N_PAD, DH), src2, dst, zacc)
  dt = degp.reshape(NW, N_PAD).T[:N]
  yh2, z2 = _tc_mid(p1.reshape(NC, N_PAD, DH), dt, z1, W_l2, W_r2,
                    b_l2.reshape(1, D))
  (p2,) = _sc_pass_nodeg(yh2.reshape(NC * N_PAD, DH), src2, dst, zacc)
  return _tc_post(p2.reshape(NC, N_PAD, DH), dt, z2)
